# Initial kernel scaffold; baseline (speedup 1.0000x reference)
import functools

import jax
import jax.numpy as jnp
from jax import lax
from jax.experimental import pallas as pl
from jax.experimental.pallas import tpu as pltpu
from jax.experimental.pallas import tpu_sc as plsc

_B = 1024
_L = 200
_D = 64
_N = _B * _L

_NC = 2
_NS = 16
_NW = _NC * _NS

_GB = 2
_GROUP_ROWS = _GB * _L
_CHUNK = 100
_NCHUNK = _GROUP_ROWS // _CHUNK
_GROUPS_PER_W = _B // (_NW * _GB)


def _body(ids_hbm, table_hbm, pos_hbm, out_hbm, idx_v, rows_v, pos_v, sem):
    c = lax.axis_index("c")
    s = lax.axis_index("s")
    wid = s * _NC + c

    pltpu.sync_copy(pos_hbm.at[pl.ds(0, _L)], pos_v)

    def group_body(g, carry):
        grp = wid * _GROUPS_PER_W + g
        pltpu.sync_copy(ids_hbm.at[pl.ds(grp * _NCHUNK, _NCHUNK)], idx_v)
        copies = []
        for ci in range(_NCHUNK):
            copies.append(
                pltpu.async_copy(
                    table_hbm.at[idx_v.at[ci]],
                    rows_v.at[pl.ds(ci * _CHUNK, _CHUNK)],
                    sem,
                )
            )
        for cp in copies:
            cp.wait()

        def add_body(r, carry2):
            for b2 in range(_GB):
                for j in range(_D // 16):
                    pv = pos_v[r, pl.ds(j * 16, 16)]
                    plsc.addupdate(rows_v.at[b2 * _L + r, pl.ds(j * 16, 16)], pv)
            return carry2

        lax.fori_loop(0, _L, add_body, 0)

        pltpu.sync_copy(rows_v, out_hbm.at[pl.ds(grp * _GROUP_ROWS, _GROUP_ROWS)])
        return carry

    lax.fori_loop(0, _GROUPS_PER_W, group_body, 0)


@jax.jit
def _embed(ids2d, token_table, pos_table):
    mesh = plsc.VectorSubcoreMesh(core_axis_name="c", subcore_axis_name="s")
    f = pl.kernel(
        _body,
        out_type=jax.ShapeDtypeStruct((_N, _D), jnp.float32),
        mesh=mesh,
        scratch_types=[
            pltpu.VMEM((_NCHUNK, _CHUNK), jnp.int32),
            pltpu.VMEM((_GROUP_ROWS, _D), jnp.float32),
            pltpu.VMEM((_L, _D), jnp.float32),
            pltpu.SemaphoreType.DMA,
        ],
    )
    return f(ids2d, token_table, pos_table)


def kernel(input_ids, token_table, pos_table):
    ids2d = input_ids.astype(jnp.int32).reshape(_N // _CHUNK, _CHUNK)
    out = _embed(ids2d, token_table, pos_table)
    return out.reshape(_B, _L, _D)


# speedup vs baseline: 1.2945x; 1.2945x over previous
import functools

import jax
import jax.numpy as jnp
from jax import lax
from jax.experimental import pallas as pl
from jax.experimental.pallas import tpu as pltpu
from jax.experimental.pallas import tpu_sc as plsc

_B = 1024
_L = 200
_D = 64
_N = _B * _L

_NC = 2
_NS = 16
_NW = _NC * _NS

_GB = 2
_GROUP_ROWS = _GB * _L
_CHUNK = 100
_NCHUNK = _GROUP_ROWS // _CHUNK
_GROUPS_PER_W = _B // (_NW * _GB)


def _body(ids_hbm, table_hbm, pos_hbm, out_hbm, idx_v, rows_v, pos_v, sem):
    c = lax.axis_index("c")
    s = lax.axis_index("s")
    wid = s * _NC + c

    pltpu.sync_copy(pos_hbm.at[pl.ds(0, _L)], pos_v)

    def group_body(g, carry):
        grp = wid * _GROUPS_PER_W + g
        pltpu.sync_copy(ids_hbm.at[pl.ds(grp * _NCHUNK, _NCHUNK)], idx_v)
        copies = []
        for ci in range(_NCHUNK):
            copies.append(
                pltpu.async_copy(
                    table_hbm.at[idx_v.at[ci]],
                    rows_v.at[pl.ds(ci * _CHUNK, _CHUNK)],
                    sem,
                )
            )
        for cp in copies:
            cp.wait()

        def add_body(r, carry2):
            for b2 in range(_GB):
                for j in range(_D // 16):
                    pv = pos_v[r, pl.ds(j * 16, 16)]
                    plsc.addupdate(rows_v.at[b2 * _L + r, pl.ds(j * 16, 16)], pv)
            return carry2

        lax.fori_loop(0, _L, add_body, 0)

        pltpu.sync_copy(rows_v, out_hbm.at[pl.ds(grp * _GROUP_ROWS, _GROUP_ROWS)])
        return carry

    lax.fori_loop(0, _GROUPS_PER_W, group_body, 0)


@jax.jit
def _embed(ids2d, token_table, pos_table):
    mesh = plsc.VectorSubcoreMesh(core_axis_name="c", subcore_axis_name="s")
    f = pl.kernel(
        _body,
        out_type=jax.ShapeDtypeStruct((_N, _D), jnp.float32),
        mesh=mesh,
        scratch_types=[
            pltpu.VMEM((_NCHUNK, _CHUNK), jnp.int32),
            pltpu.VMEM((_GROUP_ROWS, _D), jnp.float32),
            pltpu.VMEM((_L, _D), jnp.float32),
            pltpu.SemaphoreType.DMA,
        ],
        compiler_params=pltpu.CompilerParams(use_tc_tiling_on_sc=False),
    )
    return f(ids2d, token_table, pos_table)


def kernel(input_ids, token_table, pos_table):
    ids2d = input_ids.astype(jnp.int32).reshape(_N // _CHUNK, _CHUNK)
    out = _embed(ids2d, token_table, pos_table)
    return out.reshape(_B, _L, _D)


# speedup vs baseline: 1.3667x; 1.0558x over previous
import jax
import jax.numpy as jnp
from jax import lax
from jax.experimental import pallas as pl
from jax.experimental.pallas import tpu as pltpu
from jax.experimental.pallas import tpu_sc as plsc

_B = 1024
_L = 200
_D = 64
_N = _B * _L

_NC = 2
_NS = 16
_NW = _NC * _NS

_GB = 4
_GROUP_ROWS = _GB * _L
_CHUNK = 100
_NCHUNK = _GROUP_ROWS // _CHUNK
_G = _B // (_NW * _GB)


def _body(ids_hbm, table_hbm, pos_hbm, out_hbm,
          idx0, idx1, rows0, rows1, pos_v,
          isem0, isem1, gsem0, gsem1, ssem0, ssem1):
    c = lax.axis_index("c")
    s = lax.axis_index("s")
    wid = s * _NC + c

    idx = [idx0, idx1]
    rows = [rows0, rows1]
    isem = [isem0, isem1]
    gsem = [gsem0, gsem1]
    ssem = [ssem0, ssem1]

    pltpu.sync_copy(pos_hbm.at[pl.ds(0, _L)], pos_v)

    base_chunk = wid * (_G * _NCHUNK)

    def idx_copy(g):
        return pltpu.async_copy(
            ids_hbm.at[pl.ds(base_chunk + g * _NCHUNK, _NCHUNK)],
            idx[g % 2], isem[g % 2])

    def issue_gathers(g):
        hs = []
        for ci in range(_NCHUNK):
            hs.append(pltpu.async_copy(
                table_hbm.at[idx[g % 2].at[ci]],
                rows[g % 2].at[pl.ds(ci * _CHUNK, _CHUNK)],
                gsem[g % 2]))
        return hs

    ih = {0: idx_copy(0), 1: idx_copy(1)}
    ih[0].wait()
    gh = {0: issue_gathers(0)}
    sh = {}

    for g in range(_G):
        p = g % 2
        for h in gh[g]:
            h.wait()
        if g + 1 < _G:
            if g >= 1:
                sh[g - 1].wait()
            ih[g + 1].wait()
            gh[g + 1] = issue_gathers(g + 1)
            if g + 2 < _G:
                ih[g + 2] = idx_copy(g + 2)

        rv = rows[p]

        def add_body(r, carry):
            for j in range(_D // 16):
                pv = pos_v[r, pl.ds(j * 16, 16)]
                for b in range(_GB):
                    plsc.addupdate(rv.at[b * _L + r, pl.ds(j * 16, 16)], pv)
            return carry

        lax.fori_loop(0, _L, add_body, 0)

        sh[g] = pltpu.async_copy(
            rv, out_hbm.at[pl.ds((wid * _G + g) * _GROUP_ROWS, _GROUP_ROWS)],
            ssem[p])

    sh[_G - 2].wait()
    sh[_G - 1].wait()


@jax.jit
def _embed(ids2d, token_table, pos_table):
    mesh = plsc.VectorSubcoreMesh(core_axis_name="c", subcore_axis_name="s")
    f = pl.kernel(
        _body,
        out_type=jax.ShapeDtypeStruct((_N, _D), jnp.float32),
        mesh=mesh,
        scratch_types=[
            pltpu.VMEM((_NCHUNK, _CHUNK), jnp.int32),
            pltpu.VMEM((_NCHUNK, _CHUNK), jnp.int32),
            pltpu.VMEM((_GROUP_ROWS, _D), jnp.float32),
            pltpu.VMEM((_GROUP_ROWS, _D), jnp.float32),
            pltpu.VMEM((_L, _D), jnp.float32),
            pltpu.SemaphoreType.DMA,
            pltpu.SemaphoreType.DMA,
            pltpu.SemaphoreType.DMA,
            pltpu.SemaphoreType.DMA,
            pltpu.SemaphoreType.DMA,
            pltpu.SemaphoreType.DMA,
        ],
        compiler_params=pltpu.CompilerParams(use_tc_tiling_on_sc=False),
    )
    return f(ids2d, token_table, pos_table)


def kernel(input_ids, token_table, pos_table):
    ids2d = input_ids.astype(jnp.int32).reshape(_N // _CHUNK, _CHUNK)
    out = _embed(ids2d, token_table, pos_table)
    return out.reshape(_B, _L, _D)


# speedup vs baseline: 1.3671x; 1.0003x over previous
import jax
import jax.numpy as jnp
from jax import lax
from jax.experimental import pallas as pl
from jax.experimental.pallas import tpu as pltpu
from jax.experimental.pallas import tpu_sc as plsc

_B = 1024
_L = 200
_D = 64
_N = _B * _L

_NC = 2
_NS = 16
_NW = _NC * _NS

_GB = 4
_GROUP_ROWS = _GB * _L
_SPLITS = ((0, 104), (104, 96))
_G = _B // (_NW * _GB)


def _body(ids_hbm, table_hbm, pos_hbm, out_hbm,
          idx0, idx1, rows0, rows1, pos_v,
          isem0, isem1, gsem0, gsem1, ssem0, ssem1):
    c = lax.axis_index("c")
    s = lax.axis_index("s")
    wid = s * _NC + c

    idx = [idx0, idx1]
    rows = [rows0, rows1]
    isem = [isem0, isem1]
    gsem = [gsem0, gsem1]
    ssem = [ssem0, ssem1]

    pltpu.sync_copy(pos_hbm.at[pl.ds(0, _L)], pos_v)

    base_batch = wid * (_G * _GB)

    def idx_copy(g):
        return pltpu.async_copy(
            ids_hbm.at[pl.ds(base_batch + g * _GB, _GB)],
            idx[g % 2], isem[g % 2])

    def issue_gathers(g):
        hs = []
        for b in range(_GB):
            for off, size in _SPLITS:
                hs.append(pltpu.async_copy(
                    table_hbm.at[idx[g % 2].at[b, pl.ds(off, size)]],
                    rows[g % 2].at[pl.ds(b * _L + off, size)],
                    gsem[g % 2]))
        return hs

    ih = {0: idx_copy(0), 1: idx_copy(1)}
    ih[0].wait()
    gh = {0: issue_gathers(0)}
    sh = {}

    for g in range(_G):
        p = g % 2
        for h in gh[g]:
            h.wait()
        if g + 1 < _G:
            if g >= 1:
                sh[g - 1].wait()
            ih[g + 1].wait()
            gh[g + 1] = issue_gathers(g + 1)
            if g + 2 < _G:
                ih[g + 2] = idx_copy(g + 2)

        rv = rows[p]

        def add_body(r, carry):
            for j in range(_D // 16):
                pv = pos_v[r, pl.ds(j * 16, 16)]
                for b in range(_GB):
                    plsc.addupdate(rv.at[b * _L + r, pl.ds(j * 16, 16)], pv)
            return carry

        lax.fori_loop(0, _L, add_body, 0)

        sh[g] = pltpu.async_copy(
            rv, out_hbm.at[pl.ds((wid * _G + g) * _GROUP_ROWS, _GROUP_ROWS)],
            ssem[p])

    sh[_G - 2].wait()
    sh[_G - 1].wait()


@jax.jit
def _embed(ids2d, token_table, pos_table):
    mesh = plsc.VectorSubcoreMesh(core_axis_name="c", subcore_axis_name="s")
    f = pl.kernel(
        _body,
        out_type=jax.ShapeDtypeStruct((_N, _D), jnp.float32),
        mesh=mesh,
        scratch_types=[
            pltpu.VMEM((_GB, _L), jnp.int32),
            pltpu.VMEM((_GB, _L), jnp.int32),
            pltpu.VMEM((_GROUP_ROWS, _D), jnp.float32),
            pltpu.VMEM((_GROUP_ROWS, _D), jnp.float32),
            pltpu.VMEM((_L, _D), jnp.float32),
            pltpu.SemaphoreType.DMA,
            pltpu.SemaphoreType.DMA,
            pltpu.SemaphoreType.DMA,
            pltpu.SemaphoreType.DMA,
            pltpu.SemaphoreType.DMA,
            pltpu.SemaphoreType.DMA,
        ],
        compiler_params=pltpu.CompilerParams(use_tc_tiling_on_sc=False),
    )
    return f(ids2d, token_table, pos_table)


def kernel(input_ids, token_table, pos_table):
    out = _embed(input_ids.astype(jnp.int32), token_table, pos_table)
    return out.reshape(_B, _L, _D)
